# Initial kernel scaffold; baseline (speedup 1.0000x reference)
import jax
import jax.numpy as jnp
from jax.experimental import pallas as pl


def kernel(input, embed):
    raise NotImplementedError("write your pallas kernel here")


# speedup vs baseline: 1.0892x; 1.0892x over previous
import functools

import jax
import jax.numpy as jnp
from jax import lax
from jax.experimental import pallas as pl
from jax.experimental.pallas import tpu as pltpu
from jax.experimental.pallas import tpu_sc as plsc

_D = 64
_NE = 1024
_ROWS = 16 * 1024
_BM = 1024
_NB = _ROWS // _BM

_MM_PRECISION = lax.Precision.DEFAULT


def _dist_argmin_body(x_ref, e_ref, ind_ref, dsum_ref, et_ref):
    i = pl.program_id(0)
    x = x_ref[...]
    e = e_ref[...]
    a = jnp.sum(x * x, axis=1, keepdims=True)
    b = jnp.sum(e * e, axis=0, keepdims=True)
    m = jnp.dot(x, e, preferred_element_type=jnp.float32,
                precision=_MM_PRECISION)
    dist = (a - 2.0 * m) + b
    dmin = jnp.min(dist, axis=1, keepdims=True)
    col = lax.broadcasted_iota(jnp.int32, dist.shape, 1)
    ind = jnp.min(jnp.where(dist == dmin, col, jnp.int32(_NE)), axis=1)
    ind_ref[0, 0, :] = ind

    @pl.when(i == 0)
    def _():
        dsum_ref[0, 0] = 0.0
        et_ref[...] = e.T

    dsum_ref[0, 0] += jnp.sum(dmin)


_NC, _NS = 2, 16
_NW = _NC * _NS
_BPW = _ROWS // _NW
_CH = 128
_NCH = _BPW // _CH


@functools.cache
def _make_sc_gather():
    @functools.partial(
        pl.kernel,
        out_type=jax.ShapeDtypeStruct((_ROWS, _D), jnp.float32),
        mesh=plsc.VectorSubcoreMesh(core_axis_name="c", subcore_axis_name="s",
                                    num_cores=_NC, num_subcores=_NS),
        scratch_types=[
            pltpu.VMEM((_NCH, _CH), jnp.int32),
            pltpu.VMEM((_BPW, _D), jnp.float32),
            pltpu.SemaphoreType.DMA,
        ],
        compiler_params=pltpu.CompilerParams(use_tc_tiling_on_sc=False),
    )
    def _sc_gather(et_hbm, idx_hbm, out_hbm, idx_v, rows_v, sem):
        wid = lax.axis_index("s") * _NC + lax.axis_index("c")
        pltpu.sync_copy(idx_hbm.at[pl.ds(wid * _NCH, _NCH)], idx_v)
        copies = [
            pltpu.async_copy(et_hbm.at[idx_v.at[j]],
                             rows_v.at[pl.ds(j * _CH, _CH)], sem)
            for j in range(_NCH)
        ]
        for c in copies:
            c.wait()
        pltpu.sync_copy(rows_v, out_hbm.at[pl.ds(wid * _BPW, _BPW)])

    return _sc_gather


def kernel(input, embed):
    flat = input.reshape(_ROWS, _D)
    ind3, dsum, et = pl.pallas_call(
        _dist_argmin_body,
        grid=(_NB,),
        in_specs=[
            pl.BlockSpec((_BM, _D), lambda i: (i, 0)),
            pl.BlockSpec((_D, _NE), lambda i: (0, 0)),
        ],
        out_specs=[
            pl.BlockSpec((1, 1, _BM), lambda i: (i, 0, 0)),
            pl.BlockSpec(memory_space=pltpu.SMEM),
            pl.BlockSpec((_NE, _D), lambda i: (0, 0)),
        ],
        out_shape=[
            jax.ShapeDtypeStruct((_NB, 1, _BM), jnp.int32),
            jax.ShapeDtypeStruct((1, 1), jnp.float32),
            jax.ShapeDtypeStruct((_NE, _D), jnp.float32),
        ],
        compiler_params=pltpu.CompilerParams(
            dimension_semantics=("arbitrary",),
        ),
    )(flat, embed)
    q = _make_sc_gather()(et, ind3.reshape(_NW * _NCH, _CH))
    quantize = q.reshape(input.shape)
    diff = dsum[0, 0] / jnp.float32(_ROWS * _D)
    embed_ind = ind3.reshape(input.shape[:-1])
    return quantize, diff, embed_ind


# speedup vs baseline: 1.1320x; 1.0394x over previous
import functools

import jax
import jax.numpy as jnp
from jax import lax
from jax.experimental import pallas as pl
from jax.experimental.pallas import tpu as pltpu
from jax.experimental.pallas import tpu_sc as plsc

_D = 64
_NE = 1024
_ROWS = 16 * 1024
_BM = 1024
_NB = _ROWS // _BM

_NC, _NS = 2, 16
_NW = _NC * _NS
_BPW = _ROWS // _NW
_CH = 128
_NCH = _BPW // _CH

_MM_PRECISION = lax.Precision.DEFAULT


def _dist_argmin_body(x_ref, e_ref, ind_ref, ip_ref, dsum_ref, et_ref):
    i = pl.program_id(0)
    x = x_ref[...]
    e = e_ref[...]
    a = jnp.sum(x * x, axis=1, keepdims=True)
    b = jnp.sum(e * e, axis=0, keepdims=True)
    m = jnp.dot(x, e, preferred_element_type=jnp.float32,
                precision=_MM_PRECISION)
    dist = (a - 2.0 * m) + b
    dmin = jnp.min(dist, axis=1, keepdims=True)
    col = lax.broadcasted_iota(jnp.int32, dist.shape, 1)
    ind = jnp.min(jnp.where(dist == dmin, col, jnp.int32(_NE)), axis=1)
    ind_ref[0, 0, :] = ind

    ind8 = ind.reshape(8, _CH)
    z4 = jnp.zeros((4, _CH), jnp.int32)
    ip_ref[0] = jnp.concatenate([ind8[0:4], z4, ind8[4:8], z4], axis=0)

    @pl.when(i == 0)
    def _():
        dsum_ref[0, 0] = 0.0
        et_ref[...] = jnp.concatenate(
            [e.T, jnp.zeros((_NE, 128 - _D), jnp.float32)], axis=1)

    dsum_ref[0, 0] += jnp.sum(dmin)


@functools.cache
def _make_sc_gather():
    @functools.partial(
        pl.kernel,
        out_type=jax.ShapeDtypeStruct((_ROWS, 128), jnp.float32),
        mesh=plsc.VectorSubcoreMesh(core_axis_name="c", subcore_axis_name="s",
                                    num_cores=_NC, num_subcores=_NS),
        scratch_types=[
            pltpu.VMEM((8, _CH), jnp.int32),
            pltpu.VMEM((_BPW, 128), jnp.float32),
            pltpu.SemaphoreType.DMA,
        ],
    )
    def _sc_gather(et_hbm, idx_hbm, out_hbm, idx_v, rows_v, sem):
        wid = lax.axis_index("s") * _NC + lax.axis_index("c")
        pltpu.sync_copy(idx_hbm.at[pl.ds(wid * 8, 8)], idx_v)
        copies = [
            pltpu.async_copy(et_hbm.at[idx_v.at[j]],
                             rows_v.at[pl.ds(j * _CH, _CH)], sem)
            for j in range(_NCH)
        ]
        for c in copies:
            c.wait()
        pltpu.sync_copy(rows_v, out_hbm.at[pl.ds(wid * _BPW, _BPW)])

    return _sc_gather


def kernel(input, embed):
    flat = input.reshape(_ROWS, _D)
    ind3, ipad, dsum, et = pl.pallas_call(
        _dist_argmin_body,
        grid=(_NB,),
        in_specs=[
            pl.BlockSpec((_BM, _D), lambda i: (i, 0)),
            pl.BlockSpec((_D, _NE), lambda i: (0, 0)),
        ],
        out_specs=[
            pl.BlockSpec((1, 1, _BM), lambda i: (i, 0, 0)),
            pl.BlockSpec((1, 16, _CH), lambda i: (i, 0, 0)),
            pl.BlockSpec(memory_space=pltpu.SMEM),
            pl.BlockSpec((_NE, 128), lambda i: (0, 0)),
        ],
        out_shape=[
            jax.ShapeDtypeStruct((_NB, 1, _BM), jnp.int32),
            jax.ShapeDtypeStruct((_NB, 16, _CH), jnp.int32),
            jax.ShapeDtypeStruct((1, 1), jnp.float32),
            jax.ShapeDtypeStruct((_NE, 128), jnp.float32),
        ],
        compiler_params=pltpu.CompilerParams(
            dimension_semantics=("arbitrary",),
        ),
    )(flat, embed)
    q = _make_sc_gather()(et, ipad.reshape(_NB * 16, _CH))
    quantize = q[:, :_D].reshape(input.shape)
    diff = dsum[0, 0] / jnp.float32(_ROWS * _D)
    embed_ind = ind3.reshape(input.shape[:-1])
    return quantize, diff, embed_ind
